# baseline (speedup 1.0000x reference)
import functools

import jax
import jax.numpy as jnp
from jax import lax
from jax.experimental import pallas as pl
from jax.experimental.pallas import tpu as pltpu
from jax.experimental.pallas import tpu_sc as plsc


B, S, D = 4, 8192, 1024
NC, NS = 2, 16
NW = NC * NS
ROWS_PER_W = S // NW
C = 32
N_CHUNKS = ROWS_PER_W // C
CW = C * D


@functools.partial(
    pl.kernel,
    out_type=jax.ShapeDtypeStruct((B * S * D,), jnp.float32),
    mesh=plsc.VectorSubcoreMesh(core_axis_name="c", subcore_axis_name="s"),
    scratch_types=[
        pltpu.VMEM((CW,), jnp.float32),
        pltpu.VMEM((CW,), jnp.float32),
    ],
)
def _sc_add(x_hbm, pos_hbm, out_hbm, pos_v, x_v):
    wid = lax.axis_index("s") * NC + lax.axis_index("c")
    s0 = wid * ROWS_PER_W

    def chunk_body(ci, _):
        pos_off = (s0 + ci * C) * D
        pltpu.sync_copy(pos_hbm.at[pl.ds(pos_off, CW)], pos_v)

        def b_body(b, _):
            x_off = b * (S * D) + pos_off
            pltpu.sync_copy(x_hbm.at[pl.ds(x_off, CW)], x_v)

            def add_body(i, _):
                sl = pl.ds(i * 16, 16)
                x_v[sl] = x_v[sl] + pos_v[sl]
                return 0

            lax.fori_loop(0, CW // 16, add_body, 0)
            pltpu.sync_copy(x_v, out_hbm.at[pl.ds(x_off, CW)])
            return 0

        lax.fori_loop(0, B, b_body, 0)
        return 0

    lax.fori_loop(0, N_CHUNKS, chunk_body, 0)


def kernel(x, pos_emb):
    out = _sc_add(x.reshape(-1), pos_emb.reshape(-1))
    return out.reshape(x.shape)


# speedup vs baseline: 1.5843x; 1.5843x over previous
import functools

import jax
import jax.numpy as jnp
from jax import lax
from jax.experimental import pallas as pl
from jax.experimental.pallas import tpu as pltpu
from jax.experimental.pallas import tpu_sc as plsc


B, S, D = 4, 8192, 1024
NC, NS = 2, 16
NW = NC * NS
ROWS_PER_W = S // NW
C = 16
N_CHUNKS = ROWS_PER_W // C
CW = C * D
N_SL = CW // 16


@functools.partial(
    pl.kernel,
    out_type=jax.ShapeDtypeStruct((B * S * D,), jnp.float32),
    mesh=plsc.VectorSubcoreMesh(core_axis_name="c", subcore_axis_name="s"),
    scratch_types=[
        pltpu.VMEM((CW,), jnp.float32),
        pltpu.VMEM((CW,), jnp.float32),
        pltpu.VMEM((CW,), jnp.float32),
        pltpu.SemaphoreType.DMA,
        pltpu.SemaphoreType.DMA,
        pltpu.SemaphoreType.DMA,
    ],
)
def _sc_add(x_hbm, pos_hbm, out_hbm, pos_v, xb0, xb1, sx0, sx1, so):
    wid = lax.axis_index("s") * NC + lax.axis_index("c")
    s0 = wid * ROWS_PER_W
    xbufs = (xb0, xb1)
    sxs = (sx0, sx1)

    def chunk_body(ci, _):
        pos_off = (s0 + ci * C) * D
        pltpu.sync_copy(pos_hbm.at[pl.ds(pos_off, CW)], pos_v)

        def x_off(b):
            return b * (S * D) + pos_off

        in_copies = [None] * B
        out_copies = [None] * B
        in_copies[0] = pltpu.async_copy(
            x_hbm.at[pl.ds(x_off(0), CW)], xbufs[0], sxs[0])
        for b in range(B):
            buf = xbufs[b % 2]
            if b + 1 < B:
                nxt = xbufs[(b + 1) % 2]
                if b >= 1:
                    out_copies[b - 1].wait()
                in_copies[b + 1] = pltpu.async_copy(
                    x_hbm.at[pl.ds(x_off(b + 1), CW)], nxt, sxs[(b + 1) % 2])
            in_copies[b].wait()

            @plsc.parallel_loop(0, N_SL, unroll=8)
            def _add(i):
                sl = pl.ds(i * 16, 16)
                buf[sl] = buf[sl] + pos_v[sl]

            out_copies[b] = pltpu.async_copy(
                buf, out_hbm.at[pl.ds(x_off(b), CW)], so)
        out_copies[B - 2].wait()
        out_copies[B - 1].wait()
        return 0

    lax.fori_loop(0, N_CHUNKS, chunk_body, 0)


def kernel(x, pos_emb):
    out = _sc_add(x.reshape(-1), pos_emb.reshape(-1))
    return out.reshape(x.shape)


# speedup vs baseline: 1.8785x; 1.1857x over previous
import functools

import jax
import jax.numpy as jnp
from jax import lax
from jax.experimental import pallas as pl
from jax.experimental.pallas import tpu as pltpu
from jax.experimental.pallas import tpu_sc as plsc


B, S, D = 4, 8192, 1024
S_BLK = 2048

B_SC = 1
NC, NS = 2, 16
NW = NC * NS
ROWS_PER_W = B_SC * S // NW
C = 16
N_CHUNKS = ROWS_PER_W // C
CW = C * D
N_SL = CW // 16


@functools.partial(
    pl.kernel,
    out_type=jax.ShapeDtypeStruct((B_SC * S * D,), jnp.float32),
    mesh=plsc.VectorSubcoreMesh(core_axis_name="c", subcore_axis_name="s"),
    scratch_types=[
        pltpu.VMEM((CW,), jnp.float32),
        pltpu.VMEM((CW,), jnp.float32),
        pltpu.VMEM((CW,), jnp.float32),
        pltpu.SemaphoreType.DMA,
        pltpu.SemaphoreType.DMA,
        pltpu.SemaphoreType.DMA,
    ],
)
def _sc_add(x_hbm, pos_hbm, out_hbm, pos_v, xb0, xb1, sx0, sx1, so):
    wid = lax.axis_index("s") * NC + lax.axis_index("c")
    s0 = wid * ROWS_PER_W
    xbufs = (xb0, xb1)
    sxs = (sx0, sx1)

    def chunk_body(ci, _):
        pos_off = (s0 + ci * C) * D
        pltpu.sync_copy(pos_hbm.at[pl.ds(pos_off, CW)], pos_v)

        def x_off(b):
            return b * (S * D) + pos_off

        in_copies = [None] * B_SC
        out_copies = [None] * B_SC
        in_copies[0] = pltpu.async_copy(
            x_hbm.at[pl.ds(x_off(0), CW)], xbufs[0], sxs[0])
        for b in range(B_SC):
            buf = xbufs[b % 2]
            if b + 1 < B_SC:
                if b >= 1:
                    out_copies[b - 1].wait()
                in_copies[b + 1] = pltpu.async_copy(
                    x_hbm.at[pl.ds(x_off(b + 1), CW)], xbufs[(b + 1) % 2],
                    sxs[(b + 1) % 2])
            in_copies[b].wait()

            @plsc.parallel_loop(0, N_SL, unroll=8)
            def _add(i):
                sl = pl.ds(i * 16, 16)
                buf[sl] = buf[sl] + pos_v[sl]

            out_copies[b] = pltpu.async_copy(
                buf, out_hbm.at[pl.ds(x_off(b), CW)], so)
        for cp in out_copies[-2:]:
            if cp is not None:
                cp.wait()
        return 0

    lax.fori_loop(0, N_CHUNKS, chunk_body, 0)


def _tc_body(x_ref, pos_ref, o_ref):
    o_ref[...] = x_ref[...] + pos_ref[...][None, :, :]


def _tc_add(x, pos):
    n_b = B - B_SC
    n_s = S // S_BLK
    return pl.pallas_call(
        _tc_body,
        grid=(n_s, n_b),
        in_specs=[
            pl.BlockSpec((1, S_BLK, D), lambda i_s, i_b: (i_b + B_SC, i_s, 0)),
            pl.BlockSpec((S_BLK, D), lambda i_s, i_b: (i_s, 0)),
        ],
        out_specs=pl.BlockSpec((1, S_BLK, D), lambda i_s, i_b: (i_b, i_s, 0)),
        out_shape=jax.ShapeDtypeStruct((n_b, S, D), x.dtype),
        compiler_params=pltpu.CompilerParams(
            dimension_semantics=("parallel", "arbitrary"),
        ),
    )(x, pos)


def kernel(x, pos_emb):
    out_sc = _sc_add(x.reshape(-1), pos_emb.reshape(-1))
    out_tc = _tc_add(x, pos_emb)
    return jnp.concatenate([out_sc.reshape(B_SC, S, D), out_tc], axis=0)


# speedup vs baseline: 7.5736x; 4.0318x over previous
import jax
import jax.numpy as jnp
from jax.experimental import pallas as pl
from jax.experimental.pallas import tpu as pltpu


B, S, D = 4, 8192, 1024
S_BLK = 2048


def _add_body(x_ref, pos_ref, o_ref):
    o_ref[...] = x_ref[...] + pos_ref[...][None, :, :]


def kernel(x, pos_emb):
    b, s, d = x.shape
    n_s = s // S_BLK
    pos = pos_emb[:s]
    return pl.pallas_call(
        _add_body,
        grid=(n_s, b),
        in_specs=[
            pl.BlockSpec((1, S_BLK, d), lambda i_s, i_b: (i_b, i_s, 0)),
            pl.BlockSpec((S_BLK, d), lambda i_s, i_b: (i_s, 0)),
        ],
        out_specs=pl.BlockSpec((1, S_BLK, d), lambda i_s, i_b: (i_b, i_s, 0)),
        out_shape=jax.ShapeDtypeStruct((b, s, d), x.dtype),
        compiler_params=pltpu.CompilerParams(
            dimension_semantics=("parallel", "arbitrary"),
            vmem_limit_bytes=128 * 1024 * 1024,
        ),
    )(x, pos)
